# baseline (speedup 1.0000x reference)
import functools

import jax
import jax.numpy as jnp
from jax import lax
from jax.experimental import pallas as pl
from jax.experimental.pallas import tpu as pltpu
from jax.experimental.pallas import tpu_sc as plsc

EMB_DIM = 32
B = 16384
MARGIN = 1.0
NC = 2
NS = 16
L = 16
NW = NC * NS
BW = B // NW
CHUNK = 128
NCHUNK = BW // CHUNK
NGROUP = BW // L


def _rsqrt(a):
    i = plsc.bitcast(a, jnp.int32)
    i = jnp.int32(0x5F3759DF) - (i >> 1)
    y = plsc.bitcast(i, jnp.float32)
    for _ in range(3):
        y = y * (1.5 - 0.5 * a * y * y)
    return y


def _score(hh, tt, rr, hr, ht, rt):
    rh = _rsqrt(hh)
    rt_ = _rsqrt(tt)
    s2 = rr + 2.0 + 2.0 * (hr * rh - ht * (rh * rt_) - rt * rt_)
    s2 = jnp.maximum(s2, 0.0)
    return s2 * _rsqrt(s2 + 1e-30)


def _sc_body(idx_hbm, ent_hbm, rel_hbm, out_hbm,
             idx_v, ph, pt, nh, nt, pr, nr, accv, sem):
    wid = lax.axis_index("s") * NC + lax.axis_index("c")

    pltpu.sync_copy(idx_hbm.at[wid], idx_v)
    tables = [(ent_hbm, ph), (ent_hbm, pt), (ent_hbm, nh), (ent_hbm, nt),
              (rel_hbm, pr), (rel_hbm, nr)]
    copies = []
    for k, (tab, dst) in enumerate(tables):
        for j in range(NCHUNK):
            copies.append(pltpu.async_copy(
                tab.at[idx_v.at[k, j]], dst.at[pl.ds(j * CHUNK, CHUNK)], sem))
    for cp in copies:
        cp.wait()

    iota = lax.iota(jnp.int32, L)

    def group(g, acc):
        row = g * L + iota
        z = jnp.zeros((L,), jnp.float32)
        p = [z] * 6
        n = [z] * 6
        for d in range(EMB_DIM):
            col = jnp.full((L,), d, jnp.int32)
            h = plsc.load_gather(ph, [row, col])
            t = plsc.load_gather(pt, [row, col])
            r = plsc.load_gather(pr, [row, col])
            p = [p[0] + h * h, p[1] + t * t, p[2] + r * r,
                 p[3] + h * r, p[4] + h * t, p[5] + r * t]
            h = plsc.load_gather(nh, [row, col])
            t = plsc.load_gather(nt, [row, col])
            r = plsc.load_gather(nr, [row, col])
            n = [n[0] + h * h, n[1] + t * t, n[2] + r * r,
                 n[3] + h * r, n[4] + h * t, n[5] + r * t]
        ps = _score(*p)
        ns = _score(*n)
        return acc + jnp.maximum(ps - ns + MARGIN, 0.0)

    acc = lax.fori_loop(0, NGROUP, group, jnp.zeros((L,), jnp.float32))
    accv[...] = acc
    pltpu.sync_copy(accv, out_hbm.at[wid])


_sc_call = functools.partial(
    pl.kernel,
    out_type=jax.ShapeDtypeStruct((NW, L), jnp.float32),
    mesh=plsc.VectorSubcoreMesh(core_axis_name="c", subcore_axis_name="s"),
    compiler_params=pltpu.CompilerParams(needs_layout_passes=False,
                                         use_tc_tiling_on_sc=False),
    scratch_types=[
        pltpu.VMEM((6, NCHUNK, CHUNK), jnp.int32),
        pltpu.VMEM((BW, EMB_DIM), jnp.float32),
        pltpu.VMEM((BW, EMB_DIM), jnp.float32),
        pltpu.VMEM((BW, EMB_DIM), jnp.float32),
        pltpu.VMEM((BW, EMB_DIM), jnp.float32),
        pltpu.VMEM((BW, EMB_DIM), jnp.float32),
        pltpu.VMEM((BW, EMB_DIM), jnp.float32),
        pltpu.VMEM((L,), jnp.float32),
        pltpu.SemaphoreType.DMA,
    ],
)(_sc_body)


def kernel(pos_exmpls, neg_exmpls, ent_emb, rel_emb):
    idx = jnp.stack([pos_exmpls[:, 0], pos_exmpls[:, 2],
                     neg_exmpls[:, 0], neg_exmpls[:, 2],
                     pos_exmpls[:, 1], neg_exmpls[:, 1]], axis=0)
    idx = idx.reshape(6, NW, NCHUNK, CHUNK).transpose(1, 0, 2, 3)
    partial = _sc_call(idx, ent_emb, rel_emb)
    return jnp.sum(partial) / jnp.float32(B)
